# baseline (speedup 1.0000x reference)
import jax
import jax.numpy as jnp
from jax.experimental import pallas as pl
from jax.experimental.pallas import tpu as pltpu

_BN = 5000


def _fused_body(x_ref, wz_ref, wh_ref, bz_ref, bh_ref, wlin_ref, blin_ref,
                out_ref, acc_ref):
    f_half = x_ref.shape[1]
    f_out = wlin_ref.shape[1]
    j = pl.program_id(1)
    off = j * f_half
    wz = (wz_ref[0, 0, pl.ds(off, f_half), :]
          + wz_ref[1, 0, pl.ds(off, f_half), :]) * 0.5
    wh = (wh_ref[0, 0, pl.ds(off, f_half), :]
          + wh_ref[1, 0, pl.ds(off, f_half), :])
    w = jnp.concatenate([wz, wh], axis=1).astype(jnp.bfloat16)
    p = jnp.dot(x_ref[...], w, preferred_element_type=jnp.float32)

    @pl.when(j == 0)
    def _first():
        acc_ref[...] = p

    @pl.when(j == 1)
    def _last():
        b = jnp.concatenate([bz_ref[...] * 0.5, bh_ref[...]], axis=1)
        q = jnp.tanh(acc_ref[...] + p + b)
        qz = q[:, :f_out]
        qh = q[:, f_out:]
        h = jnp.maximum((1.0 - qz) * qh, 0.0)
        r = jnp.dot(0.5 * wlin_ref[...], h.T,
                    preferred_element_type=jnp.float32)
        out_ref[0] = r + blin_ref[0, 0]


def kernel(x, edge_index, edge_weight, Wz, bz, Wr, br, Wh, bh, W_lin, b_lin):
    n, f_in = x.shape
    f_out = Wz.shape[-1]
    full4 = lambda i, j: (0, 0, 0, 0)
    full = lambda i, j: (0, 0)
    wspec = pl.BlockSpec(Wz.shape, full4)
    bspec = pl.BlockSpec((1, f_out), full)
    out = pl.pallas_call(
        _fused_body,
        grid=(n // _BN, 2),
        compiler_params=pltpu.CompilerParams(
            dimension_semantics=("parallel", "arbitrary")),
        in_specs=[
            pl.BlockSpec((_BN, f_in // 2), lambda i, j: (i, j)),
            wspec, wspec, bspec, bspec,
            pl.BlockSpec((1, f_out), full),
            pl.BlockSpec((1, 1), full),
        ],
        out_specs=pl.BlockSpec((1, 1, _BN), lambda i, j: (i, 0, 0)),
        out_shape=jax.ShapeDtypeStruct((n // _BN, 1, _BN), x.dtype),
        scratch_shapes=[pltpu.VMEM((_BN, 2 * f_out), jnp.float32)],
    )(
        x, Wz, Wh,
        bz.reshape(1, f_out), bh.reshape(1, f_out),
        W_lin, b_lin.reshape(1, 1),
    )
    return out.reshape(n, 1)


# speedup vs baseline: 1.1008x; 1.1008x over previous
import jax
import jax.numpy as jnp
from jax.experimental import pallas as pl

_BN = 5000


def _fused_body(xa_ref, xb_ref, wz_ref, wh_ref, bz_ref, bh_ref, wlin_ref,
                blin_ref, out_ref):
    f_half = xa_ref.shape[1]
    f_in = 2 * f_half
    f_out = wlin_ref.shape[1]
    wz = (wz_ref[0, 0, :f_in, :] + wz_ref[1, 0, :f_in, :]) * 0.5
    wh = wh_ref[0, 0, :f_in, :] + wh_ref[1, 0, :f_in, :]
    w = jnp.concatenate([wz, wh], axis=1).astype(jnp.bfloat16)
    b = jnp.concatenate([bz_ref[...] * 0.5, bh_ref[...]], axis=1)
    p = (jnp.dot(xa_ref[...].astype(jnp.bfloat16), w[:f_half],
                 preferred_element_type=jnp.float32)
         + jnp.dot(xb_ref[...].astype(jnp.bfloat16), w[f_half:],
                   preferred_element_type=jnp.float32))
    q = jnp.tanh(p + b)
    qz = q[:, :f_out]
    qh = q[:, f_out:]
    h = jnp.maximum((1.0 - qz) * qh, 0.0)
    r = jnp.dot(0.5 * wlin_ref[...], h.T,
                preferred_element_type=jnp.float32)
    out_ref[0] = r + blin_ref[0, 0]


def kernel(x, edge_index, edge_weight, Wz, bz, Wr, br, Wh, bh, W_lin, b_lin):
    n, f_in = x.shape
    f_out = Wz.shape[-1]
    full4 = lambda i: (0, 0, 0, 0)
    full = lambda i: (0, 0)
    wspec = pl.BlockSpec(Wz.shape, full4)
    bspec = pl.BlockSpec((1, f_out), full)
    out = pl.pallas_call(
        _fused_body,
        grid=(n // _BN,),
        in_specs=[
            pl.BlockSpec((_BN, f_in // 2), lambda i: (i, 0)),
            pl.BlockSpec((_BN, f_in // 2), lambda i: (i, 1)),
            wspec, wspec, bspec, bspec,
            pl.BlockSpec((1, f_out), full),
            pl.BlockSpec((1, 1), full),
        ],
        out_specs=pl.BlockSpec((1, 1, _BN), lambda i: (i, 0, 0)),
        out_shape=jax.ShapeDtypeStruct((n // _BN, 1, _BN), x.dtype),
    )(
        x, x, Wz, Wh,
        bz.reshape(1, f_out), bh.reshape(1, f_out),
        W_lin, b_lin.reshape(1, 1),
    )
    return out.reshape(n, 1)
